# baseline (speedup 1.0000x reference)
import functools

import jax
import jax.numpy as jnp
from jax import lax
from jax.experimental import pallas as pl
from jax.experimental.pallas import tpu as pltpu
from jax.experimental.pallas import tpu_sc as plsc

_N_BOOKS = 16
_K = 256
_D = 32


def _codes_kernel(x_ref, cbm2_ref, c2t_ref, codes_ref, flat_ref):
    x = x_ref[...]
    bt = x.shape[0]
    iota0 = jax.lax.broadcasted_iota(jnp.int32, (_K, bt), 0)
    code_rows = []
    flat_rows = []
    for n in range(_N_BOOKS):
        xn = x[:, n * _D:(n + 1) * _D]
        cross_t = jax.lax.dot_general(
            cbm2_ref[n], xn, (((1,), (1,)), ((), ())),
            preferred_element_type=jnp.float32)
        score = c2t_ref[:, n:n + 1] + cross_t
        minval = jnp.min(score, axis=0, keepdims=True)
        idx = jnp.min(jnp.where(score == minval, iota0, _K), axis=0,
                      keepdims=True)
        code_rows.append(idx)
        flat_rows.append(idx + n * _K)
    codes_t = jnp.concatenate(code_rows, axis=0)
    codes_ref[...] = codes_t.T.astype(jnp.uint8)
    flat_ref[...] = jnp.concatenate(flat_rows, axis=0)


def _sc_gather(flat_hbm, table_hbm, out_hbm, idx_v, rows_v, sem):
    nc = 2
    wid = lax.axis_index("s") * nc + lax.axis_index("c")
    book = wid // 2
    half = wid % 2
    pltpu.sync_copy(flat_hbm.at[book, pl.ds(half * 64, 64)], idx_v)

    def body(j, carry):
        pltpu.async_copy(table_hbm.at[idx_v.at[j]], rows_v, sem).wait()
        pltpu.sync_copy(
            rows_v, out_hbm.at[pl.ds(half * 8192 + j * 128, 128), book])
        return carry

    lax.fori_loop(0, 64, body, 0)


@jax.jit
def _vq(x, codebooks):
    b, e = x.shape
    block_b = 1024
    cbm2 = -2.0 * codebooks
    c2t = jnp.sum(codebooks * codebooks, axis=-1).T
    codes, flat = pl.pallas_call(
        _codes_kernel,
        grid=(b // block_b,),
        in_specs=[
            pl.BlockSpec((block_b, e), lambda i: (i, 0)),
            pl.BlockSpec((_N_BOOKS, _K, _D), lambda i: (0, 0, 0)),
            pl.BlockSpec((_K, _N_BOOKS), lambda i: (0, 0)),
        ],
        out_specs=[
            pl.BlockSpec((block_b, _N_BOOKS), lambda i: (i, 0)),
            pl.BlockSpec((_N_BOOKS, block_b), lambda i: (0, i)),
        ],
        out_shape=[
            jax.ShapeDtypeStruct((b, _N_BOOKS), jnp.uint8),
            jax.ShapeDtypeStruct((_N_BOOKS, b), jnp.int32),
        ],
    )(x, cbm2, c2t)

    flat3 = flat.reshape(_N_BOOKS, b // 128, 128)
    table = codebooks.reshape(_N_BOOKS * _K, _D)
    mesh = plsc.VectorSubcoreMesh(core_axis_name="c", subcore_axis_name="s")
    recon3 = functools.partial(
        pl.kernel,
        mesh=mesh,
        compiler_params=pltpu.CompilerParams(use_tc_tiling_on_sc=False),
        out_type=jax.ShapeDtypeStruct((b, _N_BOOKS, _D), jnp.float32),
        scratch_types=[
            pltpu.VMEM((64, 128), jnp.int32),
            pltpu.VMEM((128, _D), jnp.float32),
            pltpu.SemaphoreType.DMA,
        ],
    )(_sc_gather)(flat3, table)
    return codes, recon3.reshape(b, e)


def kernel(x, codebooks):
    return _vq(x, codebooks)


# speedup vs baseline: 1.7307x; 1.7307x over previous
import functools

import jax
import jax.numpy as jnp
from jax.experimental import pallas as pl
from jax.experimental.pallas import tpu as pltpu

_N_BOOKS = 16
_K = 256
_D = 32


def _vq_block_kernel(x_ref, cbm2_ref, cbhl_ref, c2t_ref, codes_ref, recon_ref):
    x = x_ref[...]
    bt = x.shape[0]
    iota0 = jax.lax.broadcasted_iota(jnp.int32, (_K, bt), 0)
    code_rows = []
    recon_cols = []
    for n in range(_N_BOOKS):
        xn = x[:, n * _D:(n + 1) * _D]
        cross_t = jax.lax.dot_general(
            cbm2_ref[n], xn, (((1,), (1,)), ((), ())),
            preferred_element_type=jnp.float32)
        score = c2t_ref[:, n:n + 1] + cross_t
        minval = jnp.min(score, axis=0, keepdims=True)
        idx = jnp.min(jnp.where(score == minval, iota0, _K), axis=0,
                      keepdims=True)
        onehot = (iota0 == idx).astype(jnp.float32)
        rec2 = jax.lax.dot_general(
            onehot, cbhl_ref[n], (((0,), (0,)), ((), ())),
            preferred_element_type=jnp.float32)
        code_rows.append(idx)
        recon_cols.append(rec2[:, :_D] + rec2[:, _D:])
    codes_t = jnp.concatenate(code_rows, axis=0)
    codes_ref[...] = codes_t.T.astype(jnp.uint8)
    recon_ref[...] = jnp.concatenate(recon_cols, axis=1)


@jax.jit
def _vq(x, codebooks):
    b, e = x.shape
    block_b = 1024
    cbm2 = -2.0 * codebooks
    cb_hi = codebooks.astype(jnp.bfloat16).astype(jnp.float32)
    cbhl = jnp.concatenate([cb_hi, codebooks - cb_hi], axis=-1)
    c2t = jnp.sum(codebooks * codebooks, axis=-1).T
    return pl.pallas_call(
        _vq_block_kernel,
        grid=(b // block_b,),
        in_specs=[
            pl.BlockSpec((block_b, e), lambda i: (i, 0)),
            pl.BlockSpec((_N_BOOKS, _K, _D), lambda i: (0, 0, 0)),
            pl.BlockSpec((_N_BOOKS, _K, 2 * _D), lambda i: (0, 0, 0)),
            pl.BlockSpec((_K, _N_BOOKS), lambda i: (0, 0)),
        ],
        out_specs=[
            pl.BlockSpec((block_b, _N_BOOKS), lambda i: (i, 0)),
            pl.BlockSpec((block_b, e), lambda i: (i, 0)),
        ],
        out_shape=[
            jax.ShapeDtypeStruct((b, _N_BOOKS), jnp.uint8),
            jax.ShapeDtypeStruct((b, e), jnp.float32),
        ],
    )(x, cbm2, cbhl, c2t)


def kernel(x, codebooks):
    return _vq(x, codebooks)
